# initial kernel scaffold (unmeasured)
import jax
import jax.numpy as jnp
from jax import lax
from jax.experimental import pallas as pl
from jax.experimental.pallas import tpu as pltpu

N_DEV = 16


def kernel(x, w_mat):
    x = x.astype(jnp.bfloat16)
    w = w_mat.astype(jnp.bfloat16)
    m_per, k = x.shape
    n = w.shape[1]

    def body(x_ref, w_ref, out_ref, gather_ref, send_sems, recv_sems,
             amax_ref, a_send, a_recv):
        me = lax.axis_index("i")
        right = lax.rem(me + 1, N_DEV)

        gather_ref[0] = x_ref[...]
        y = jnp.dot(x_ref[...], w_ref[...],
                    preferred_element_type=jnp.float32)
        run_max = jnp.max(y)
        out_ref[pl.ds(me * m_per, m_per), :] = y

        for h in range(N_DEV - 1):
            rdma = pltpu.make_async_remote_copy(
                src_ref=gather_ref.at[h],
                dst_ref=gather_ref.at[h + 1],
                send_sem=send_sems.at[h],
                recv_sem=recv_sems.at[h + 1],
                device_id=(right,),
                device_id_type=pl.DeviceIdType.MESH,
            )
            rdma.start()
            rdma.wait()
            origin = lax.rem(me + N_DEV - h - 1, N_DEV)
            y = jnp.dot(gather_ref[h + 1], w_ref[...],
                        preferred_element_type=jnp.float32)
            run_max = jnp.maximum(run_max, jnp.max(y))
            out_ref[pl.ds(origin * m_per, m_per), :] = y

        amax_ref[0] = jnp.full((8, 128), jnp.maximum(run_max, 0.0),
                               jnp.float32)
        rdmas = []
        for d in range(1, N_DEV):
            tgt = lax.rem(me + d, N_DEV)
            r = pltpu.make_async_remote_copy(
                src_ref=amax_ref.at[0],
                dst_ref=amax_ref.at[d],
                send_sem=a_send.at[d],
                recv_sem=a_recv.at[d],
                device_id=(tgt,),
                device_id_type=pl.DeviceIdType.MESH,
            )
            r.start()
            rdmas.append(r)
        for r in rdmas:
            r.wait_recv()
        for r in rdmas:
            r.wait_send()

        amax = jnp.max(amax_ref[...])
        scale = amax / 127.0
        inv = 127.0 / amax
        for b in range(N_DEV):
            yb = jnp.maximum(out_ref[pl.ds(b * m_per, m_per), :], 0.0)
            q = jnp.clip(jnp.round(yb * inv), 0.0, 127.0)
            out_ref[pl.ds(b * m_per, m_per), :] = q * scale

    return pl.pallas_call(
        body,
        out_shape=jax.ShapeDtypeStruct((N_DEV * m_per, n), jnp.float32),
        in_specs=[
            pl.BlockSpec(memory_space=pltpu.VMEM),
            pl.BlockSpec(memory_space=pltpu.VMEM),
        ],
        out_specs=pl.BlockSpec(memory_space=pltpu.VMEM),
        scratch_shapes=[
            pltpu.VMEM((N_DEV, m_per, k), jnp.bfloat16),
            pltpu.SemaphoreType.DMA((N_DEV,)),
            pltpu.SemaphoreType.DMA((N_DEV,)),
            pltpu.VMEM((N_DEV, 8, 128), jnp.float32),
            pltpu.SemaphoreType.DMA((N_DEV,)),
            pltpu.SemaphoreType.DMA((N_DEV,)),
        ],
    )(x, w)


# baseline (device time: 425710 ns/iter reference)
import jax
import jax.numpy as jnp
from jax import lax
from jax.experimental import pallas as pl
from jax.experimental.pallas import tpu as pltpu

N_DEV = 16


def kernel(x, w_mat):
    x = x.astype(jnp.bfloat16)
    w = w_mat.astype(jnp.bfloat16)
    m_per, k = x.shape
    n = w.shape[1]

    def body(x_ref, w_ref, out_ref, gather_ref, send_sems, recv_sems,
             amax_ref, a_send, a_recv):
        me = lax.axis_index("i")
        right = lax.rem(me + 1, N_DEV)

        gather_ref[0] = x_ref[...]
        y = jnp.dot(x_ref[...], w_ref[...],
                    preferred_element_type=jnp.float32)
        run_max = jnp.max(y)
        out_ref[pl.ds(me * m_per, m_per), :] = y

        for h in range(N_DEV - 1):
            rdma = pltpu.make_async_remote_copy(
                src_ref=gather_ref.at[h],
                dst_ref=gather_ref.at[h + 1],
                send_sem=send_sems.at[h],
                recv_sem=recv_sems.at[h + 1],
                device_id=(right,),
                device_id_type=pl.DeviceIdType.MESH,
            )
            rdma.start()
            rdma.wait()
            origin = lax.rem(me + N_DEV - h - 1, N_DEV)
            y = jnp.dot(gather_ref[h + 1], w_ref[...],
                        preferred_element_type=jnp.float32)
            run_max = jnp.maximum(run_max, jnp.max(y))
            out_ref[pl.ds(origin * m_per, m_per), :] = y

        amax_ref[0] = jnp.full((8, 128), jnp.maximum(run_max, 0.0),
                               jnp.float32)
        rdmas = []
        for d in range(1, N_DEV):
            tgt = lax.rem(me + d, N_DEV)
            r = pltpu.make_async_remote_copy(
                src_ref=amax_ref.at[0],
                dst_ref=amax_ref.at[d],
                send_sem=a_send.at[d],
                recv_sem=a_recv.at[d],
                device_id=(tgt,),
                device_id_type=pl.DeviceIdType.MESH,
            )
            r.start()
            rdmas.append(r)
        for r in rdmas:
            r.wait_recv()
        for r in rdmas:
            r.wait_send()

        amax = jnp.max(amax_ref[...])
        scale = amax / 127.0
        inv = 127.0 / amax
        for b in range(N_DEV):
            yb = jnp.maximum(out_ref[pl.ds(b * m_per, m_per), :], 0.0)
            q = jnp.clip(jnp.round(yb * inv), 0.0, 127.0)
            out_ref[pl.ds(b * m_per, m_per), :] = q * scale

    return pl.pallas_call(
        body,
        out_shape=jax.ShapeDtypeStruct((N_DEV * m_per, n), jnp.float32),
        in_specs=[
            pl.BlockSpec(memory_space=pltpu.VMEM),
            pl.BlockSpec(memory_space=pltpu.VMEM),
        ],
        out_specs=pl.BlockSpec(memory_space=pltpu.VMEM),
        scratch_shapes=[
            pltpu.VMEM((N_DEV, m_per, k), jnp.bfloat16),
            pltpu.SemaphoreType.DMA((N_DEV,)),
            pltpu.SemaphoreType.DMA((N_DEV,)),
            pltpu.VMEM((N_DEV, 8, 128), jnp.float32),
            pltpu.SemaphoreType.DMA((N_DEV,)),
            pltpu.SemaphoreType.DMA((N_DEV,)),
        ],
        compiler_params=pltpu.CompilerParams(
            vmem_limit_bytes=56 * 1024 * 1024,
        ),
    )(x, w)


# device time: 210225 ns/iter; 2.0250x vs baseline; 2.0250x over previous
import jax
import jax.numpy as jnp
from jax import lax
from jax.experimental import pallas as pl
from jax.experimental.pallas import tpu as pltpu

N_DEV = 16
NSUB = 2


def kernel(x, w_mat):
    x = x.astype(jnp.bfloat16)
    w = w_mat.astype(jnp.bfloat16)
    m_per, k = x.shape
    n = w.shape[1]
    half = m_per // 2
    sub = half // NSUB

    def body(x_ref, w_ref, out_ref, f_ref, b_ref,
             fs, fr, bs, br, amax_ref, a_send, a_recv):
        me = lax.axis_index("i")
        right = lax.rem(me + 1, N_DEV)
        left = lax.rem(me + N_DEV - 1, N_DEV)

        def ring_rdma(src, dst, ssem, rsem, tgt):
            return pltpu.make_async_remote_copy(
                src_ref=src, dst_ref=dst, send_sem=ssem, recv_sem=rsem,
                device_id=(tgt,), device_id_type=pl.DeviceIdType.MESH,
            )

        for s in range(NSUB):
            rows = pl.ds(s * sub, sub)
            ring_rdma(x_ref.at[rows], f_ref.at[1, rows],
                      fs.at[0, s], fr.at[1, s], right).start()
            ring_rdma(x_ref.at[pl.ds(half + s * sub, sub)], b_ref.at[1, rows],
                      bs.at[0, s], br.at[1, s], left).start()

        y = jnp.dot(x_ref[...], w_ref[...],
                    preferred_element_type=jnp.float32)
        run_max0 = jnp.max(y)
        out_ref[pl.ds(me * m_per, m_per), :] = y

        def hop(h, run_max):
            for s in range(NSUB):
                rows = pl.ds(s * sub, sub)
                ring_rdma(f_ref.at[h, rows], f_ref.at[h, rows],
                          fs.at[h, s], fr.at[h, s], right).wait_recv()

                @pl.when(h < N_DEV - 1)
                def _():
                    ring_rdma(f_ref.at[h, rows], f_ref.at[h + 1, rows],
                              fs.at[h, s], fr.at[h + 1, s], right).start()

                ring_rdma(b_ref.at[h, rows], b_ref.at[h, rows],
                          bs.at[h, s], br.at[h, s], left).wait_recv()

                @pl.when(h < N_DEV - 1)
                def _():
                    ring_rdma(b_ref.at[h, rows], b_ref.at[h + 1, rows],
                              bs.at[h, s], br.at[h + 1, s], left).start()

            of = lax.rem(me + N_DEV - h, N_DEV)
            ob = lax.rem(me + h, N_DEV)
            yf = jnp.dot(f_ref[h], w_ref[...],
                         preferred_element_type=jnp.float32)
            out_ref[pl.ds(of * m_per, half), :] = yf
            yb = jnp.dot(b_ref[h], w_ref[...],
                         preferred_element_type=jnp.float32)
            out_ref[pl.ds(ob * m_per + half, half), :] = yb
            return jnp.maximum(run_max,
                               jnp.maximum(jnp.max(yf), jnp.max(yb)))

        run_max = lax.fori_loop(1, N_DEV, hop, run_max0, unroll=False)

        amax_ref[0] = jnp.full((8, 128), jnp.maximum(run_max, 0.0),
                               jnp.float32)
        rdmas = []
        for d in range(1, N_DEV):
            tgt = lax.rem(me + d, N_DEV)
            r = pltpu.make_async_remote_copy(
                src_ref=amax_ref.at[0], dst_ref=amax_ref.at[d],
                send_sem=a_send.at[d], recv_sem=a_recv.at[d],
                device_id=(tgt,), device_id_type=pl.DeviceIdType.MESH,
            )
            r.start()
            rdmas.append(r)
        for r in rdmas:
            r.wait_recv()

        def drain(h, c):
            for s in range(NSUB):
                rows = pl.ds(s * sub, sub)
                ring_rdma(f_ref.at[1, rows], f_ref.at[1, rows],
                          fs.at[h, s], fr.at[1, s], right).wait_send()
                ring_rdma(b_ref.at[1, rows], b_ref.at[1, rows],
                          bs.at[h, s], br.at[1, s], left).wait_send()
            return c

        lax.fori_loop(0, N_DEV - 1, drain, 0, unroll=False)
        for r in rdmas:
            r.wait_send()

        amax = jnp.max(amax_ref[...])
        scale = amax / 127.0
        inv = 127.0 / amax

        def quant_block(bi, c):
            yb = jnp.maximum(out_ref[pl.ds(bi * m_per, m_per), :], 0.0)
            q = jnp.clip(jnp.round(yb * inv), 0.0, 127.0)
            out_ref[pl.ds(bi * m_per, m_per), :] = q * scale
            return c

        lax.fori_loop(0, N_DEV, quant_block, 0, unroll=False)

    return pl.pallas_call(
        body,
        out_shape=jax.ShapeDtypeStruct((N_DEV * m_per, n), jnp.float32),
        in_specs=[
            pl.BlockSpec(memory_space=pltpu.VMEM),
            pl.BlockSpec(memory_space=pltpu.VMEM),
        ],
        out_specs=pl.BlockSpec(memory_space=pltpu.VMEM),
        scratch_shapes=[
            pltpu.VMEM((N_DEV, half, k), jnp.bfloat16),
            pltpu.VMEM((N_DEV, half, k), jnp.bfloat16),
            pltpu.SemaphoreType.DMA((N_DEV, NSUB)),
            pltpu.SemaphoreType.DMA((N_DEV, NSUB)),
            pltpu.SemaphoreType.DMA((N_DEV, NSUB)),
            pltpu.SemaphoreType.DMA((N_DEV, NSUB)),
            pltpu.VMEM((N_DEV, 8, 128), jnp.float32),
            pltpu.SemaphoreType.DMA((N_DEV,)),
            pltpu.SemaphoreType.DMA((N_DEV,)),
        ],
        compiler_params=pltpu.CompilerParams(
            vmem_limit_bytes=56 * 1024 * 1024,
        ),
    )(x, w)


# device time: 207789 ns/iter; 2.0488x vs baseline; 1.0117x over previous
import jax
import jax.numpy as jnp
from jax import lax
from jax.experimental import pallas as pl
from jax.experimental.pallas import tpu as pltpu

N_DEV = 16
NSUB = 4


def kernel(x, w_mat):
    x = x.astype(jnp.bfloat16)
    w = w_mat.astype(jnp.bfloat16)
    m_per, k = x.shape
    n = w.shape[1]
    half = m_per // 2
    sub = half // NSUB

    def body(x_ref, w_ref, out_ref, f_ref, b_ref,
             fs, fr, bs, br, amax_ref, a_send, a_recv):
        me = lax.axis_index("i")
        right = lax.rem(me + 1, N_DEV)
        left = lax.rem(me + N_DEV - 1, N_DEV)

        barrier_sem = pltpu.get_barrier_semaphore()
        for nbr in (left, right):
            pl.semaphore_signal(barrier_sem, inc=1, device_id=(nbr,),
                                device_id_type=pl.DeviceIdType.MESH)
        pl.semaphore_wait(barrier_sem, 2)

        def ring_rdma(src, dst, ssem, rsem, tgt):
            return pltpu.make_async_remote_copy(
                src_ref=src, dst_ref=dst, send_sem=ssem, recv_sem=rsem,
                device_id=(tgt,), device_id_type=pl.DeviceIdType.MESH,
            )

        for s in range(NSUB):
            rows = pl.ds(s * sub, sub)
            ring_rdma(x_ref.at[rows], f_ref.at[1, rows],
                      fs.at[0, s], fr.at[1, s], right).start()
            ring_rdma(x_ref.at[pl.ds(half + s * sub, sub)], b_ref.at[1, rows],
                      bs.at[0, s], br.at[1, s], left).start()

        y = jnp.dot(x_ref[...], w_ref[...],
                    preferred_element_type=jnp.float32)
        run_max0 = jnp.max(y)
        out_ref[pl.ds(me * m_per, m_per), :] = y

        def hop(h, run_max):
            for s in range(NSUB):
                rows = pl.ds(s * sub, sub)
                ring_rdma(f_ref.at[h, rows], f_ref.at[h, rows],
                          fs.at[h, s], fr.at[h, s], right).wait_recv()

                @pl.when(h < N_DEV - 1)
                def _():
                    ring_rdma(f_ref.at[h, rows], f_ref.at[h + 1, rows],
                              fs.at[h, s], fr.at[h + 1, s], right).start()

                ring_rdma(b_ref.at[h, rows], b_ref.at[h, rows],
                          bs.at[h, s], br.at[h, s], left).wait_recv()

                @pl.when(h < N_DEV - 1)
                def _():
                    ring_rdma(b_ref.at[h, rows], b_ref.at[h + 1, rows],
                              bs.at[h, s], br.at[h + 1, s], left).start()

            of = lax.rem(me + N_DEV - h, N_DEV)
            ob = lax.rem(me + h, N_DEV)
            yf = jnp.dot(f_ref[h], w_ref[...],
                         preferred_element_type=jnp.float32)
            out_ref[pl.ds(of * m_per, half), :] = yf
            yb = jnp.dot(b_ref[h], w_ref[...],
                         preferred_element_type=jnp.float32)
            out_ref[pl.ds(ob * m_per + half, half), :] = yb
            return jnp.maximum(run_max,
                               jnp.maximum(jnp.max(yf), jnp.max(yb)))

        run_max = lax.fori_loop(1, N_DEV, hop, run_max0, unroll=False)

        amax_ref[0] = jnp.full((8, 128), jnp.maximum(run_max, 0.0),
                               jnp.float32)
        rdmas = []
        for d in range(1, N_DEV):
            tgt = lax.rem(me + d, N_DEV)
            r = pltpu.make_async_remote_copy(
                src_ref=amax_ref.at[0], dst_ref=amax_ref.at[d],
                send_sem=a_send.at[d], recv_sem=a_recv.at[d],
                device_id=(tgt,), device_id_type=pl.DeviceIdType.MESH,
            )
            r.start()
            rdmas.append(r)
        for r in rdmas:
            r.wait_recv()

        def drain(h, c):
            for s in range(NSUB):
                rows = pl.ds(s * sub, sub)
                ring_rdma(f_ref.at[1, rows], f_ref.at[1, rows],
                          fs.at[h, s], fr.at[1, s], right).wait_send()
                ring_rdma(b_ref.at[1, rows], b_ref.at[1, rows],
                          bs.at[h, s], br.at[1, s], left).wait_send()
            return c

        lax.fori_loop(0, N_DEV - 1, drain, 0, unroll=False)
        for r in rdmas:
            r.wait_send()

        amax = jnp.max(amax_ref[...])
        scale = amax / 127.0
        inv = 127.0 / amax

        def quant_block(bi, c):
            yb = jnp.maximum(out_ref[pl.ds(bi * m_per, m_per), :], 0.0)
            q = jnp.clip(jnp.round(yb * inv), 0.0, 127.0)
            out_ref[pl.ds(bi * m_per, m_per), :] = q * scale
            return c

        lax.fori_loop(0, N_DEV, quant_block, 0, unroll=False)

    return pl.pallas_call(
        body,
        out_shape=jax.ShapeDtypeStruct((N_DEV * m_per, n), jnp.float32),
        in_specs=[
            pl.BlockSpec(memory_space=pltpu.VMEM),
            pl.BlockSpec(memory_space=pltpu.VMEM),
        ],
        out_specs=pl.BlockSpec(memory_space=pltpu.VMEM),
        scratch_shapes=[
            pltpu.VMEM((N_DEV, half, k), jnp.bfloat16),
            pltpu.VMEM((N_DEV, half, k), jnp.bfloat16),
            pltpu.SemaphoreType.DMA((N_DEV, NSUB)),
            pltpu.SemaphoreType.DMA((N_DEV, NSUB)),
            pltpu.SemaphoreType.DMA((N_DEV, NSUB)),
            pltpu.SemaphoreType.DMA((N_DEV, NSUB)),
            pltpu.VMEM((N_DEV, 8, 128), jnp.float32),
            pltpu.SemaphoreType.DMA((N_DEV,)),
            pltpu.SemaphoreType.DMA((N_DEV,)),
        ],
        compiler_params=pltpu.CompilerParams(
            vmem_limit_bytes=56 * 1024 * 1024,
            collective_id=0,
        ),
    )(x, w)


# device time: 191682 ns/iter; 2.2209x vs baseline; 1.0840x over previous
import jax
import jax.numpy as jnp
from jax import lax
from jax.experimental import pallas as pl
from jax.experimental.pallas import tpu as pltpu

N_DEV = 16
NSUB = 4


def kernel(x, w_mat):
    m_per, k = x.shape
    n = w_mat.shape[1]
    half = m_per // 2
    sub = half // NSUB

    def body(x_hbm, w_hbm, out_ref, f_ref, b_ref,
             fs, fr, bs, br, amax_ref, a_send, a_recv,
             wbf_ref, obf_ref, out_sems, x_ref, y_ref, in_sems):
        me = lax.axis_index("i")
        right = lax.rem(me + 1, N_DEV)
        left = lax.rem(me + N_DEV - 1, N_DEV)

        cp_x = pltpu.make_async_copy(x_hbm, x_ref, in_sems.at[0])
        cp_x.start()
        cp_w = pltpu.make_async_copy(w_hbm, y_ref, in_sems.at[1])
        cp_w.start()

        barrier_sem = pltpu.get_barrier_semaphore()
        for nbr in (left, right):
            pl.semaphore_signal(barrier_sem, inc=1, device_id=(nbr,),
                                device_id_type=pl.DeviceIdType.MESH)
        pl.semaphore_wait(barrier_sem, 2)
        cp_x.wait()

        def ring_rdma(src, dst, ssem, rsem, tgt):
            return pltpu.make_async_remote_copy(
                src_ref=src, dst_ref=dst, send_sem=ssem, recv_sem=rsem,
                device_id=(tgt,), device_id_type=pl.DeviceIdType.MESH,
            )

        f_ref[0] = x_ref[pl.ds(0, half), :].astype(jnp.bfloat16)
        b_ref[0] = x_ref[pl.ds(half, half), :].astype(jnp.bfloat16)
        for s in range(NSUB):
            rows = pl.ds(s * sub, sub)
            ring_rdma(f_ref.at[0, rows], f_ref.at[1, rows],
                      fs.at[0, s], fr.at[1, s], right).start()
            ring_rdma(b_ref.at[0, rows], b_ref.at[1, rows],
                      bs.at[0, s], br.at[1, s], left).start()

        cp_w.wait()
        for c in range(4):
            rows = pl.ds(c * (k // 4), k // 4)
            wbf_ref[rows, :] = y_ref[rows, :].astype(jnp.bfloat16)

        yf0 = jnp.dot(f_ref[0], wbf_ref[...],
                      preferred_element_type=jnp.float32)
        y_ref[pl.ds(me * m_per, half), :] = yf0
        yb0 = jnp.dot(b_ref[0], wbf_ref[...],
                      preferred_element_type=jnp.float32)
        y_ref[pl.ds(me * m_per + half, half), :] = yb0
        run_max0 = jnp.maximum(jnp.max(yf0), jnp.max(yb0))

        def hop(h, run_max):
            for s in range(NSUB):
                rows = pl.ds(s * sub, sub)
                ring_rdma(f_ref.at[h, rows], f_ref.at[h, rows],
                          fs.at[h, s], fr.at[h, s], right).wait_recv()

                @pl.when(h < N_DEV - 1)
                def _():
                    ring_rdma(f_ref.at[h, rows], f_ref.at[h + 1, rows],
                              fs.at[h, s], fr.at[h + 1, s], right).start()

                ring_rdma(b_ref.at[h, rows], b_ref.at[h, rows],
                          bs.at[h, s], br.at[h, s], left).wait_recv()

                @pl.when(h < N_DEV - 1)
                def _():
                    ring_rdma(b_ref.at[h, rows], b_ref.at[h + 1, rows],
                              bs.at[h, s], br.at[h + 1, s], left).start()

            of = lax.rem(me + N_DEV - h, N_DEV)
            ob = lax.rem(me + h, N_DEV)
            yf = jnp.dot(f_ref[h], wbf_ref[...],
                         preferred_element_type=jnp.float32)
            y_ref[pl.ds(of * m_per, half), :] = yf
            yb = jnp.dot(b_ref[h], wbf_ref[...],
                         preferred_element_type=jnp.float32)
            y_ref[pl.ds(ob * m_per + half, half), :] = yb
            return jnp.maximum(run_max,
                               jnp.maximum(jnp.max(yf), jnp.max(yb)))

        run_max = lax.fori_loop(1, N_DEV, hop, run_max0, unroll=False)

        amax_ref[0] = jnp.full((8, 128), jnp.maximum(run_max, 0.0),
                               jnp.float32)
        rdmas = []
        for d in range(1, N_DEV):
            tgt = lax.rem(me + d, N_DEV)
            r = pltpu.make_async_remote_copy(
                src_ref=amax_ref.at[0], dst_ref=amax_ref.at[d],
                send_sem=a_send.at[d], recv_sem=a_recv.at[d],
                device_id=(tgt,), device_id_type=pl.DeviceIdType.MESH,
            )
            r.start()
            rdmas.append(r)
        for r in rdmas:
            r.wait_recv()

        def drain(h, c):
            for s in range(NSUB):
                rows = pl.ds(s * sub, sub)
                ring_rdma(f_ref.at[1, rows], f_ref.at[1, rows],
                          fs.at[h, s], fr.at[1, s], right).wait_send()
                ring_rdma(b_ref.at[1, rows], b_ref.at[1, rows],
                          bs.at[h, s], br.at[1, s], left).wait_send()
            return c

        lax.fori_loop(0, N_DEV - 1, drain, 0, unroll=False)
        for r in rdmas:
            r.wait_send()

        amax = jnp.max(amax_ref[...])
        scale = amax / 127.0
        inv = 127.0 / amax

        def quant_block(bi, c):
            blk = pl.ds(bi * m_per, m_per)
            q = jnp.clip(jnp.round(y_ref[blk, :] * inv), 0.0, 127.0)
            obf_ref[blk, :] = (q * scale).astype(jnp.bfloat16)
            pltpu.make_async_copy(
                obf_ref.at[blk], out_ref.at[blk], out_sems.at[bi]
            ).start()
            return c

        lax.fori_loop(0, N_DEV, quant_block, 0, unroll=False)

        def wait_out(bi, c):
            blk = pl.ds(bi * m_per, m_per)
            pltpu.make_async_copy(
                obf_ref.at[blk], out_ref.at[blk], out_sems.at[bi]
            ).wait()
            return c

        lax.fori_loop(0, N_DEV, wait_out, 0, unroll=False)

    return pl.pallas_call(
        body,
        out_shape=jax.ShapeDtypeStruct((N_DEV * m_per, n), jnp.bfloat16),
        in_specs=[
            pl.BlockSpec(memory_space=pl.ANY),
            pl.BlockSpec(memory_space=pl.ANY),
        ],
        out_specs=pl.BlockSpec(memory_space=pl.ANY),
        scratch_shapes=[
            pltpu.VMEM((N_DEV, half, k), jnp.bfloat16),
            pltpu.VMEM((N_DEV, half, k), jnp.bfloat16),
            pltpu.SemaphoreType.DMA((N_DEV, NSUB)),
            pltpu.SemaphoreType.DMA((N_DEV, NSUB)),
            pltpu.SemaphoreType.DMA((N_DEV, NSUB)),
            pltpu.SemaphoreType.DMA((N_DEV, NSUB)),
            pltpu.VMEM((N_DEV, 8, 128), jnp.float32),
            pltpu.SemaphoreType.DMA((N_DEV,)),
            pltpu.SemaphoreType.DMA((N_DEV,)),
            pltpu.VMEM((k, n), jnp.bfloat16),
            pltpu.VMEM((N_DEV * m_per, n), jnp.bfloat16),
            pltpu.SemaphoreType.DMA((N_DEV,)),
            pltpu.VMEM((m_per, k), jnp.float32),
            pltpu.VMEM((k, n), jnp.float32),
            pltpu.SemaphoreType.DMA((2,)),
        ],
        compiler_params=pltpu.CompilerParams(
            vmem_limit_bytes=60 * 1024 * 1024,
            collective_id=0,
        ),
    )(x, w_mat)
